# baseline (speedup 1.0000x reference)
import dataclasses
import functools

import jax
import jax.numpy as jnp
from jax import lax
from jax.experimental import pallas as pl
from jax.experimental.pallas import tpu as pltpu
from jax.experimental.pallas import tpu_sc as plsc

HIST_BINS = 64
VMIN = 0.0
VMAX = 255.0
L = 16
NWORKERS = 32
CH = 32768
NBANKS = 8


def _make_hist_kernel(B, NPIX):
    batches_per_w = B // NWORKERS
    nch = NPIX // CH
    mesh = plsc.VectorSubcoreMesh(core_axis_name="c", subcore_axis_name="s")
    cp = pltpu.CompilerParams()
    if "needs_layout_passes" in pltpu.CompilerParams.__dataclass_fields__:
        cp = dataclasses.replace(cp, needs_layout_passes=False)

    @functools.partial(
        pl.kernel,
        compiler_params=cp,
        out_type=jax.ShapeDtypeStruct((B, HIST_BINS), jnp.float32),
        mesh=mesh,
        scratch_types=[
            pltpu.VMEM((CH,), jnp.float32),
            pltpu.VMEM((CH,), jnp.float32),
            pltpu.VMEM((NBANKS * L * HIST_BINS,), jnp.float32),
            pltpu.VMEM((HIST_BINS,), jnp.float32),
            pltpu.SemaphoreType.DMA,
            pltpu.SemaphoreType.DMA,
        ],
    )
    def hist_kernel(gm_hbm, hist_hbm, buf0, buf1, hist16, row, sem0, sem1):
        wid = lax.axis_index("s") * 2 + lax.axis_index("c")
        lane_base = lax.iota(jnp.int32, L) * HIST_BINS
        ones = jnp.ones((L,), jnp.float32)
        zeros = jnp.zeros((L,), jnp.float32)

        def accumulate(buf):
            @pl.loop(0, CH, step=L * NBANKS)
            def _(v):
                for u in range(NBANKS):
                    x = buf[pl.ds(v + u * L, L)]
                    scaled = x / jnp.float32(VMAX - VMIN) * jnp.float32(HIST_BINS)
                    idx = scaled.astype(jnp.int32)
                    idx = jnp.clip(idx, 0, HIST_BINS - 1)
                    mask = (x >= jnp.float32(VMIN)) & (x <= jnp.float32(VMAX))
                    plsc.addupdate_scatter(
                        hist16, [lane_base + (idx + u * (L * HIST_BINS))], ones, mask=mask
                    )

        for b in range(batches_per_w):
            batch = wid * batches_per_w + b

            @pl.loop(0, NBANKS * L * HIST_BINS, step=L)
            def _(i):
                hist16[pl.ds(i, L)] = zeros

            pltpu.async_copy(gm_hbm.at[batch, pl.ds(0, CH)], buf0, sem0)
            pltpu.async_copy(gm_hbm.at[batch, pl.ds(CH, CH)], buf1, sem1)

            @pl.loop(0, nch, step=2)
            def _(c):
                pltpu.make_async_copy(gm_hbm.at[batch, pl.ds(0, CH)], buf0, sem0).wait()
                accumulate(buf0)

                @pl.when(c + 2 < nch)
                def _():
                    pltpu.async_copy(
                        gm_hbm.at[batch, pl.ds((c + 2) * CH, CH)], buf0, sem0
                    )

                pltpu.make_async_copy(gm_hbm.at[batch, pl.ds(0, CH)], buf1, sem1).wait()
                accumulate(buf1)

                @pl.when(c + 3 < nch)
                def _():
                    pltpu.async_copy(
                        gm_hbm.at[batch, pl.ds((c + 3) * CH, CH)], buf1, sem1
                    )

            for kc in range(HIST_BINS // L):
                acc = jnp.zeros((L,), jnp.float32)
                for u in range(NBANKS):
                    for l in range(L):
                        acc = acc + hist16[
                            pl.ds(u * L * HIST_BINS + l * HIST_BINS + kc * L, L)
                        ]
                row[pl.ds(kc * L, L)] = acc
            pltpu.sync_copy(row, hist_hbm.at[batch])

    return hist_kernel


def _mlp_kernel(hist_ref, w1_ref, b1_ref, w2_ref, b2_ref, out_ref):
    hist = hist_ref[...]
    norm = jnp.maximum(jnp.sum(jnp.abs(hist), axis=-1, keepdims=True), 1e-12)
    h = hist / norm
    h1 = lax.dot_general(
        h, w1_ref[...], (((1,), (1,)), ((), ())), preferred_element_type=jnp.float32
    )
    h1 = jnp.maximum(h1 + b1_ref[...], 0.0)
    out = lax.dot_general(
        h1, w2_ref[...], (((1,), (1,)), ((), ())), preferred_element_type=jnp.float32
    )
    out_ref[...] = out + b2_ref[...]


@jax.jit
def kernel(grad_map, W1, b1, W2, b2):
    B = grad_map.shape[0]
    NPIX = grad_map.shape[1] * grad_map.shape[2]
    gm = grad_map.reshape(B, NPIX)
    hist = _make_hist_kernel(B, NPIX)(gm)
    out_dim = W2.shape[0]
    out = pl.pallas_call(
        _mlp_kernel,
        out_shape=jax.ShapeDtypeStruct((B, out_dim), jnp.float32),
    )(hist, W1, b1.reshape(1, -1), W2, b2.reshape(1, -1))
    return out


# speedup vs baseline: 1.1014x; 1.1014x over previous
import dataclasses
import functools

import jax
import jax.numpy as jnp
from jax import lax
from jax.experimental import pallas as pl
from jax.experimental.pallas import tpu as pltpu
from jax.experimental.pallas import tpu_sc as plsc

HIST_BINS = 64
VMIN = 0.0
VMAX = 255.0
L = 16
NWORKERS = 32
CH = 32768
NBANKS = 8


def _make_hist_kernel(B, NPIX):
    batches_per_w = B // NWORKERS
    nch = NPIX // CH
    mesh = plsc.VectorSubcoreMesh(core_axis_name="c", subcore_axis_name="s")
    cp = pltpu.CompilerParams()
    if "needs_layout_passes" in pltpu.CompilerParams.__dataclass_fields__:
        cp = dataclasses.replace(cp, needs_layout_passes=False)

    @functools.partial(
        pl.kernel,
        compiler_params=cp,
        out_type=jax.ShapeDtypeStruct((B, HIST_BINS), jnp.float32),
        mesh=mesh,
        scratch_types=[
            pltpu.VMEM((CH,), jnp.float32),
            pltpu.VMEM((CH,), jnp.float32),
            pltpu.VMEM((NBANKS * L * HIST_BINS,), jnp.float32),
            pltpu.VMEM((HIST_BINS,), jnp.float32),
            pltpu.SemaphoreType.DMA,
            pltpu.SemaphoreType.DMA,
        ],
    )
    def hist_kernel(gm_hbm, hist_hbm, buf0, buf1, hist16, row, sem0, sem1):
        wid = lax.axis_index("s") * 2 + lax.axis_index("c")
        lane_base = lax.iota(jnp.int32, L) * HIST_BINS
        ones = jnp.ones((L,), jnp.float32)
        zeros = jnp.zeros((L,), jnp.float32)

        scale = jnp.float32(HIST_BINS / (VMAX - VMIN))
        lane_bases = [lane_base + u * (L * HIST_BINS) for u in range(NBANKS)]

        def accumulate(buf):
            @pl.loop(0, CH, step=L * NBANKS)
            def _(v):
                for u in range(NBANKS):
                    x = buf[pl.ds(v + u * L, L)]
                    idx = jnp.clip((x * scale).astype(jnp.int32), 0, HIST_BINS - 1)
                    mask = (x >= jnp.float32(VMIN)) & (x <= jnp.float32(VMAX))
                    plsc.addupdate_scatter(
                        hist16, [lane_bases[u] + idx], ones, mask=mask
                    )

        for b in range(batches_per_w):
            batch = wid * batches_per_w + b

            @pl.loop(0, NBANKS * L * HIST_BINS, step=L)
            def _(i):
                hist16[pl.ds(i, L)] = zeros

            pltpu.async_copy(gm_hbm.at[batch, pl.ds(0, CH)], buf0, sem0)
            pltpu.async_copy(gm_hbm.at[batch, pl.ds(CH, CH)], buf1, sem1)

            @pl.loop(0, nch, step=2)
            def _(c):
                pltpu.make_async_copy(gm_hbm.at[batch, pl.ds(0, CH)], buf0, sem0).wait()
                accumulate(buf0)

                @pl.when(c + 2 < nch)
                def _():
                    pltpu.async_copy(
                        gm_hbm.at[batch, pl.ds((c + 2) * CH, CH)], buf0, sem0
                    )

                pltpu.make_async_copy(gm_hbm.at[batch, pl.ds(0, CH)], buf1, sem1).wait()
                accumulate(buf1)

                @pl.when(c + 3 < nch)
                def _():
                    pltpu.async_copy(
                        gm_hbm.at[batch, pl.ds((c + 3) * CH, CH)], buf1, sem1
                    )

            for kc in range(HIST_BINS // L):
                acc = jnp.zeros((L,), jnp.float32)
                for u in range(NBANKS):
                    for l in range(L):
                        acc = acc + hist16[
                            pl.ds(u * L * HIST_BINS + l * HIST_BINS + kc * L, L)
                        ]
                row[pl.ds(kc * L, L)] = acc
            pltpu.sync_copy(row, hist_hbm.at[batch])

    return hist_kernel


def _mlp_kernel(hist_ref, w1_ref, b1_ref, w2_ref, b2_ref, out_ref):
    hist = hist_ref[...]
    norm = jnp.maximum(jnp.sum(jnp.abs(hist), axis=-1, keepdims=True), 1e-12)
    h = hist / norm
    h1 = lax.dot_general(
        h, w1_ref[...], (((1,), (1,)), ((), ())), preferred_element_type=jnp.float32
    )
    h1 = jnp.maximum(h1 + b1_ref[...], 0.0)
    out = lax.dot_general(
        h1, w2_ref[...], (((1,), (1,)), ((), ())), preferred_element_type=jnp.float32
    )
    out_ref[...] = out + b2_ref[...]


@jax.jit
def kernel(grad_map, W1, b1, W2, b2):
    B = grad_map.shape[0]
    NPIX = grad_map.shape[1] * grad_map.shape[2]
    gm = grad_map.reshape(B, NPIX)
    hist = _make_hist_kernel(B, NPIX)(gm)
    out_dim = W2.shape[0]
    out = pl.pallas_call(
        _mlp_kernel,
        out_shape=jax.ShapeDtypeStruct((B, out_dim), jnp.float32),
    )(hist, W1, b1.reshape(1, -1), W2, b2.reshape(1, -1))
    return out
